# baseline (speedup 1.0000x reference)
import functools
import jax
import jax.numpy as jnp
from jax import lax
from jax.experimental import pallas as pl
from jax.experimental.pallas import tpu as pltpu
from jax.experimental.pallas import tpu_sc as plsc

_N = 10000
_E = 320000
_NCORE = 2
_NSUB = 16
_NW = _NCORE * _NSUB
_EPT = _E // _NW
_CH = 250
_NCHUNK = _EPT // _CH
_CS_NSTAGE = 2
_RCH = 128
_NR = 5

_B = 10000
_G = _N // _B


def _fill(ref, rows, d, val):
  v = jnp.full((16,), val, jnp.float32)

  @pl.loop(0, rows)
  def _(r):
    for c in range(d // 16):
      ref[r, pl.ds(c * 16, 16)] = v


def _stripe(s, t):
  off = jnp.minimum((s * _NR + t) * _RCH, _N - _RCH)
  return pl.multiple_of(off, _RCH)


@functools.cache
def _agg(d):
  ch, nc = _CH, _NCHUNK
  mesh = plsc.VectorSubcoreMesh(core_axis_name="c", subcore_axis_name="s")

  @functools.partial(
      pl.kernel,
      out_type=jax.ShapeDtypeStruct((_NCORE, _N, d), jnp.float32),
      mesh=mesh,
      compiler_params=pltpu.CompilerParams(use_tc_tiling_on_sc=False),
      scratch_types=(
          [pltpu.VMEM((nc, ch), jnp.int32),
           pltpu.VMEM((nc, ch), jnp.int32),
           pltpu.VMEM((max(ch, _RCH), d), jnp.float32)]
          + [pltpu.VMEM((ch, d), jnp.float32)] * 3
          + [pltpu.VMEM_SHARED((_N, d), jnp.float32)]
          + [pltpu.SemaphoreType.DMA] * 8
      ),
  )
  def k(g_hbm, src_hbm, dst_hbm, out_hbm, src_v, dst_v, buf, *rest):
    rest = list(rest)
    bufs = [buf.at[pl.ds(0, ch)]] + rest[:3]
    acc = rest[3]
    gs = rest[4:8]
    ss = rest[8:12]
    c = lax.axis_index("c")
    s = lax.axis_index("s")
    wid = c * _NSUB + s
    zs = buf.at[pl.ds(0, _RCH)]
    _fill(buf, _RCH, d, 0.0)
    for t in range(_NR):
      pltpu.sync_copy(zs, acc.at[pl.ds(_stripe(s, t), _RCH)])
    plsc.subcore_barrier()

    def gather(j, r):
      pltpu.async_copy(g_hbm.at[src_v.at[j]], bufs[r], gs[r])

    def gwait(j, r):
      pltpu.make_async_copy(g_hbm.at[src_v.at[j]], bufs[r], gs[r]).wait()

    def scat(j, r):
      pltpu.async_copy(bufs[r], acc.at[dst_v.at[j]], ss[r], add=True)

    def swait(j, r):
      pltpu.make_async_copy(bufs[r], acc.at[dst_v.at[j]], ss[r]).wait()

    pltpu.sync_copy(src_hbm.at[wid], src_v)
    pltpu.sync_copy(dst_hbm.at[wid], dst_v)
    gather(0, 0)
    gather(1, 1)
    gwait(0, 0); scat(0, 0); gather(2, 2)
    gwait(1, 1); scat(1, 1); gather(3, 3)
    gwait(2, 2); scat(2, 2); swait(0, 0); gather(4, 0)
    gwait(3, 3); scat(3, 3); swait(1, 1); gather(5, 1)

    @pl.loop(1, nc // 4)
    def _(bi):
      jb = bi * 4
      for ph in range(4):
        j = jb + ph
        r2 = (ph + 2) % 4
        gwait(j, ph)
        scat(j, ph)
        swait(j - 2, r2)
        nxt = jnp.minimum(j + 2, nc - 1)
        gather(nxt, r2)

    swait(nc - 2, 2)
    swait(nc - 1, 3)
    gwait(nc - 1, 0)
    gwait(nc - 1, 1)

    plsc.subcore_barrier()
    for t in range(_NR):
      rows = pl.ds(_stripe(s, t), _RCH)
      pltpu.sync_copy(acc.at[rows], zs)
      pltpu.sync_copy(zs, out_hbm.at[c, rows])

  return k


@functools.cache
def _agg128cs():
  ch, schunk = _CH, _NCHUNK
  mesh = plsc.VectorSubcoreMesh(core_axis_name="c", subcore_axis_name="s")

  @functools.partial(
      pl.kernel,
      out_type=jax.ShapeDtypeStruct((_NCORE, _N, 64), jnp.float32),
      mesh=mesh,
      compiler_params=pltpu.CompilerParams(use_tc_tiling_on_sc=False),
      scratch_types=(
          [pltpu.VMEM((schunk, ch), jnp.int32),
           pltpu.VMEM((schunk, ch), jnp.int32),
           pltpu.VMEM((ch, 64), jnp.float32)]
          + [pltpu.VMEM((ch, 64), jnp.float32)] * 3
          + [pltpu.VMEM_SHARED((_N, 64), jnp.float32)]
          + [pltpu.SemaphoreType.DMA] * 8
      ),
  )
  def k(g_hbm, src_hbm, dst_hbm, out_hbm, src_v, dst_v, buf, *rest):
    rest = list(rest)
    bufs = [buf] + rest[:3]
    acc = rest[3]
    gs = rest[4:8]
    ss = rest[8:12]
    c = lax.axis_index("c")
    s = lax.axis_index("s")
    g_c = g_hbm.at[c]
    zs = buf.at[pl.ds(0, _RCH)]
    _fill(buf, _RCH, 64, 0.0)
    for t in range(_NR):
      pltpu.sync_copy(zs, acc.at[pl.ds(_stripe(s, t), _RCH)])
    plsc.subcore_barrier()

    def gather(j, r):
      pltpu.async_copy(g_c.at[src_v.at[j]], bufs[r], gs[r])

    def gwait(j, r):
      pltpu.make_async_copy(g_c.at[src_v.at[j]], bufs[r], gs[r]).wait()

    def scat(j, r):
      pltpu.async_copy(bufs[r], acc.at[dst_v.at[j]], ss[r], add=True)

    def swait(j, r):
      pltpu.make_async_copy(bufs[r], acc.at[dst_v.at[j]], ss[r]).wait()

    for h in range(_CS_NSTAGE):
      pltpu.sync_copy(src_hbm.at[s + _NSUB * h], src_v)
      pltpu.sync_copy(dst_hbm.at[s + _NSUB * h], dst_v)
      gather(0, 0)
      gather(1, 1)
      gwait(0, 0); scat(0, 0); gather(2, 2)
      gwait(1, 1); scat(1, 1); gather(3, 3)
      gwait(2, 2); scat(2, 2); swait(0, 0); gather(4, 0)
      gwait(3, 3); scat(3, 3); swait(1, 1); gather(5, 1)

      @pl.loop(1, schunk // 4)
      def _(bi):
        jb = bi * 4
        for ph in range(4):
          j = jb + ph
          r2 = (ph + 2) % 4
          gwait(j, ph)
          scat(j, ph)
          swait(j - 2, r2)
          nxt = jnp.minimum(j + 2, schunk - 1)
          gather(nxt, r2)

      swait(schunk - 2, 2)
      swait(schunk - 1, 3)
      gwait(schunk - 1, 0)
      gwait(schunk - 1, 1)

    plsc.subcore_barrier()
    for t in range(_NR):
      rows = pl.ds(_stripe(s, t), _RCH)
      pltpu.sync_copy(acc.at[rows], zs)
      pltpu.sync_copy(zs, out_hbm.at[c, rows])

  return k


_DW = 16


@functools.cache
def _deg():
  mesh = plsc.VectorSubcoreMesh(core_axis_name="c", subcore_axis_name="s")

  @functools.partial(
      pl.kernel,
      out_type=jax.ShapeDtypeStruct((_NCORE, _N, _DW), jnp.float32),
      mesh=mesh,
      compiler_params=pltpu.CompilerParams(use_tc_tiling_on_sc=False),
      scratch_types=[
          pltpu.VMEM((_NCHUNK, _CH), jnp.int32),
          pltpu.VMEM((max(_CH, _RCH), _DW), jnp.float32),
          pltpu.VMEM_SHARED((_N, _DW), jnp.float32),
          pltpu.SemaphoreType.DMA,
      ],
  )
  def k(dst_hbm, out_hbm, dst_v, buf, acc, sem):
    c = lax.axis_index("c")
    s = lax.axis_index("s")
    wid = c * _NSUB + s
    zs = buf.at[pl.ds(0, _RCH)]
    _fill(buf, _RCH, _DW, 0.0)
    for t in range(_NR):
      pltpu.sync_copy(zs, acc.at[pl.ds(_stripe(s, t), _RCH)])
    plsc.subcore_barrier()
    _fill(buf, _CH, _DW, 1.0)
    ones = buf.at[pl.ds(0, _CH)]
    pltpu.sync_copy(dst_hbm.at[wid], dst_v)

    @pl.loop(0, _NCHUNK)
    def _(j):
      pltpu.async_copy(ones, acc.at[dst_v.at[j]], sem, add=True)

    @pl.loop(0, _NCHUNK)
    def _(j):
      pltpu.make_async_copy(ones, acc.at[dst_v.at[j]], sem).wait()

    plsc.subcore_barrier()
    for t in range(_NR):
      rows = pl.ds(_stripe(s, t), _RCH)
      pltpu.sync_copy(acc.at[rows], zs)
      pltpu.sync_copy(zs, out_hbm.at[c, rows])

  return k


def _dinv(deg_ref):
  return lax.rsqrt(deg_ref[0, :, 0:1] + deg_ref[1, :, 0:1] + 1.0)


def _split2(z, o_ref):
  o_ref[0] = z[:, :64]
  o_ref[1] = z[:, 64:]


def _cat2(s_ref, g_ref):
  return jnp.concatenate([s_ref[0] + g_ref[0], s_ref[1] + g_ref[1]], axis=1)


def _k_g1(deg_ref, x_ref, o_ref):
  _split2(_dinv(deg_ref) * x_ref[...], o_ref)


def _k_first(deg_ref, s_ref, g_ref, w1_ref, b1_ref, w2_ref, o_ref):
  dinv = _dinv(deg_ref)
  p = dinv * _cat2(s_ref, g_ref)
  h = jnp.maximum(
      jnp.dot(p, w1_ref[...], preferred_element_type=jnp.float32) + b1_ref[...], 0.0)
  _split2(dinv * jnp.dot(h, w2_ref[...], preferred_element_type=jnp.float32), o_ref)


def _k_mid3(deg_ref, s_ref, g_ref, b_ref, w_ref, o_ref):
  dinv = _dinv(deg_ref)
  h = jnp.maximum(dinv * _cat2(s_ref, g_ref) + b_ref[...], 0.0)
  o_ref[...] = dinv * jnp.dot(h, w_ref[...], preferred_element_type=jnp.float32)


def _k_mid(deg_ref, s_ref, g_ref, b_ref, w_ref, o_ref):
  dinv = _dinv(deg_ref)
  h = jnp.maximum(dinv * (s_ref[0] + s_ref[1] + g_ref[...]) + b_ref[...], 0.0)
  o_ref[...] = dinv * jnp.dot(h, w_ref[...], preferred_element_type=jnp.float32)


def _k_last(deg_ref, s_ref, g_ref, b_ref, o_ref):
  dinv = _dinv(deg_ref)
  o_ref[...] = jnp.maximum(dinv * (s_ref[0] + s_ref[1] + g_ref[...]) + b_ref[...], 0.0)


_DEG_SPEC = pl.BlockSpec((2, _B, _DW), lambda i: (0, i, 0))


def _s_spec(p):
  return pl.BlockSpec((2, _B, p), lambda i: (0, i, 0))


def _g_spec(p):
  return pl.BlockSpec((_B, p), lambda i: (i, 0))


def _w_spec(p, q):
  return pl.BlockSpec((p, q), lambda i: (0, 0))


def _b_spec(q):
  return pl.BlockSpec((1, q), lambda i: (0, 0))


_SG_SPEC = pl.BlockSpec((2, _B, 64), lambda i: (0, i, 0))


def _pc(body, q, in_specs):
  return pl.pallas_call(
      body,
      grid=(_G,),
      in_specs=in_specs,
      out_specs=pl.BlockSpec((_B, q), lambda i: (i, 0)),
      out_shape=jax.ShapeDtypeStruct((_N, q), jnp.float32),
  )


def _pc2(body, in_specs):
  return pl.pallas_call(
      body,
      grid=(_G,),
      in_specs=in_specs,
      out_specs=_SG_SPEC,
      out_shape=jax.ShapeDtypeStruct((2, _N, 64), jnp.float32),
  )


@jax.jit
def kernel(x, edge_index, W1, b1, W2, b2, W3, b3, W4, b4, W5, b5):
  src = edge_index[0].reshape(_NW, _NCHUNK, _CH)
  dst = edge_index[1].reshape(_NW, _NCHUNK, _CH)
  ecs = (src, dst)

  deg = _deg()(dst)
  g1 = _pc2(_k_g1, [_DEG_SPEC, _g_spec(128)])(deg, x)
  S = _agg128cs()(g1, *ecs)
  g2 = _pc2(_k_first,
            [_DEG_SPEC, _SG_SPEC, _SG_SPEC, _w_spec(128, 256),
             _b_spec(256), _w_spec(256, 128)])(
                deg, S, g1, W1, b1.reshape(1, -1), W2)
  S = _agg128cs()(g2, *ecs)
  g3 = _pc(_k_mid3, 64,
           [_DEG_SPEC, _SG_SPEC, _SG_SPEC, _b_spec(128),
            _w_spec(128, 64)])(deg, S, g2, b2.reshape(1, -1), W3)
  S = _agg(64)(g3, src, dst)
  g4 = _pc(_k_mid, 32,
           [_DEG_SPEC, _s_spec(64), _g_spec(64), _b_spec(64),
            _w_spec(64, 32)])(deg, S, g3, b3.reshape(1, -1), W4)
  S = _agg(32)(g4, src, dst)
  g5 = _pc(_k_mid, 16,
           [_DEG_SPEC, _s_spec(32), _g_spec(32), _b_spec(32),
            _w_spec(32, 16)])(deg, S, g4, b4.reshape(1, -1), W5)
  S = _agg(16)(g5, src, dst)
  out = _pc(_k_last, 16, [_DEG_SPEC, _s_spec(16), _g_spec(16), _b_spec(16)])(
      deg, S, g5, b5.reshape(1, -1))
  return out


# speedup vs baseline: 1.0152x; 1.0152x over previous
import functools
import jax
import jax.numpy as jnp
from jax import lax
from jax.experimental import pallas as pl
from jax.experimental.pallas import tpu as pltpu
from jax.experimental.pallas import tpu_sc as plsc

_N = 10000
_E = 320000
_NCORE = 2
_NSUB = 16
_NW = _NCORE * _NSUB
_EPT = _E // _NW
_CH = 250
_NCHUNK = _EPT // _CH
_CS_NSTAGE = 2
_RCH = 128
_NR = 5

_B = 2000
_G = _N // _B


def _fill(ref, rows, d, val):
  v = jnp.full((16,), val, jnp.float32)

  @pl.loop(0, rows)
  def _(r):
    for c in range(d // 16):
      ref[r, pl.ds(c * 16, 16)] = v


def _stripe(s, t):
  off = jnp.minimum((s * _NR + t) * _RCH, _N - _RCH)
  return pl.multiple_of(off, _RCH)


@functools.cache
def _agg(d):
  ch, nc = _CH, _NCHUNK
  mesh = plsc.VectorSubcoreMesh(core_axis_name="c", subcore_axis_name="s")

  @functools.partial(
      pl.kernel,
      out_type=jax.ShapeDtypeStruct((_NCORE, _N, d), jnp.float32),
      mesh=mesh,
      compiler_params=pltpu.CompilerParams(use_tc_tiling_on_sc=False),
      scratch_types=(
          [pltpu.VMEM((nc, ch), jnp.int32),
           pltpu.VMEM((nc, ch), jnp.int32),
           pltpu.VMEM((max(ch, _RCH), d), jnp.float32)]
          + [pltpu.VMEM((ch, d), jnp.float32)] * 3
          + [pltpu.VMEM_SHARED((_N, d), jnp.float32)]
          + [pltpu.SemaphoreType.DMA] * 8
      ),
  )
  def k(g_hbm, src_hbm, dst_hbm, out_hbm, src_v, dst_v, buf, *rest):
    rest = list(rest)
    bufs = [buf.at[pl.ds(0, ch)]] + rest[:3]
    acc = rest[3]
    gs = rest[4:8]
    ss = rest[8:12]
    c = lax.axis_index("c")
    s = lax.axis_index("s")
    wid = c * _NSUB + s
    zs = buf.at[pl.ds(0, _RCH)]
    _fill(buf, _RCH, d, 0.0)
    for t in range(_NR):
      pltpu.sync_copy(zs, acc.at[pl.ds(_stripe(s, t), _RCH)])
    plsc.subcore_barrier()

    def gather(j, r):
      pltpu.async_copy(g_hbm.at[src_v.at[j]], bufs[r], gs[r])

    def gwait(j, r):
      pltpu.make_async_copy(g_hbm.at[src_v.at[j]], bufs[r], gs[r]).wait()

    def scat(j, r):
      pltpu.async_copy(bufs[r], acc.at[dst_v.at[j]], ss[r], add=True)

    def swait(j, r):
      pltpu.make_async_copy(bufs[r], acc.at[dst_v.at[j]], ss[r]).wait()

    pltpu.sync_copy(src_hbm.at[wid], src_v)
    pltpu.sync_copy(dst_hbm.at[wid], dst_v)
    gather(0, 0)
    gather(1, 1)
    gwait(0, 0); scat(0, 0); gather(2, 2)
    gwait(1, 1); scat(1, 1); gather(3, 3)
    gwait(2, 2); scat(2, 2); swait(0, 0); gather(4, 0)
    gwait(3, 3); scat(3, 3); swait(1, 1); gather(5, 1)

    @pl.loop(1, nc // 4)
    def _(bi):
      jb = bi * 4
      for ph in range(4):
        j = jb + ph
        r2 = (ph + 2) % 4
        gwait(j, ph)
        scat(j, ph)
        swait(j - 2, r2)
        nxt = jnp.minimum(j + 2, nc - 1)
        gather(nxt, r2)

    swait(nc - 2, 2)
    swait(nc - 1, 3)
    gwait(nc - 1, 0)
    gwait(nc - 1, 1)

    plsc.subcore_barrier()
    for t in range(_NR):
      rows = pl.ds(_stripe(s, t), _RCH)
      pltpu.sync_copy(acc.at[rows], zs)
      pltpu.sync_copy(zs, out_hbm.at[c, rows])

  return k


@functools.cache
def _agg128cs():
  ch, schunk = _CH, _NCHUNK
  mesh = plsc.VectorSubcoreMesh(core_axis_name="c", subcore_axis_name="s")

  @functools.partial(
      pl.kernel,
      out_type=jax.ShapeDtypeStruct((_NCORE, _N, 64), jnp.float32),
      mesh=mesh,
      compiler_params=pltpu.CompilerParams(use_tc_tiling_on_sc=False),
      scratch_types=(
          [pltpu.VMEM((schunk, ch), jnp.int32),
           pltpu.VMEM((schunk, ch), jnp.int32),
           pltpu.VMEM((ch, 64), jnp.float32)]
          + [pltpu.VMEM((ch, 64), jnp.float32)] * 3
          + [pltpu.VMEM_SHARED((_N, 64), jnp.float32)]
          + [pltpu.SemaphoreType.DMA] * 8
      ),
  )
  def k(g_hbm, src_hbm, dst_hbm, out_hbm, src_v, dst_v, buf, *rest):
    rest = list(rest)
    bufs = [buf] + rest[:3]
    acc = rest[3]
    gs = rest[4:8]
    ss = rest[8:12]
    c = lax.axis_index("c")
    s = lax.axis_index("s")
    g_c = g_hbm.at[c]
    zs = buf.at[pl.ds(0, _RCH)]
    _fill(buf, _RCH, 64, 0.0)
    for t in range(_NR):
      pltpu.sync_copy(zs, acc.at[pl.ds(_stripe(s, t), _RCH)])
    plsc.subcore_barrier()

    def gather(j, r):
      pltpu.async_copy(g_c.at[src_v.at[j]], bufs[r], gs[r])

    def gwait(j, r):
      pltpu.make_async_copy(g_c.at[src_v.at[j]], bufs[r], gs[r]).wait()

    def scat(j, r):
      pltpu.async_copy(bufs[r], acc.at[dst_v.at[j]], ss[r], add=True)

    def swait(j, r):
      pltpu.make_async_copy(bufs[r], acc.at[dst_v.at[j]], ss[r]).wait()

    for h in range(_CS_NSTAGE):
      pltpu.sync_copy(src_hbm.at[s + _NSUB * h], src_v)
      pltpu.sync_copy(dst_hbm.at[s + _NSUB * h], dst_v)
      gather(0, 0)
      gather(1, 1)
      gwait(0, 0); scat(0, 0); gather(2, 2)
      gwait(1, 1); scat(1, 1); gather(3, 3)
      gwait(2, 2); scat(2, 2); swait(0, 0); gather(4, 0)
      gwait(3, 3); scat(3, 3); swait(1, 1); gather(5, 1)

      @pl.loop(1, schunk // 4)
      def _(bi):
        jb = bi * 4
        for ph in range(4):
          j = jb + ph
          r2 = (ph + 2) % 4
          gwait(j, ph)
          scat(j, ph)
          swait(j - 2, r2)
          nxt = jnp.minimum(j + 2, schunk - 1)
          gather(nxt, r2)

      swait(schunk - 2, 2)
      swait(schunk - 1, 3)
      gwait(schunk - 1, 0)
      gwait(schunk - 1, 1)

    plsc.subcore_barrier()
    for t in range(_NR):
      rows = pl.ds(_stripe(s, t), _RCH)
      pltpu.sync_copy(acc.at[rows], zs)
      pltpu.sync_copy(zs, out_hbm.at[c, rows])

  return k


_DW = 16


@functools.cache
def _deg():
  mesh = plsc.VectorSubcoreMesh(core_axis_name="c", subcore_axis_name="s")

  @functools.partial(
      pl.kernel,
      out_type=jax.ShapeDtypeStruct((_NCORE, _N, _DW), jnp.float32),
      mesh=mesh,
      compiler_params=pltpu.CompilerParams(use_tc_tiling_on_sc=False),
      scratch_types=[
          pltpu.VMEM((_NCHUNK, _CH), jnp.int32),
          pltpu.VMEM((max(_CH, _RCH), _DW), jnp.float32),
          pltpu.VMEM_SHARED((_N, _DW), jnp.float32),
          pltpu.SemaphoreType.DMA,
      ],
  )
  def k(dst_hbm, out_hbm, dst_v, buf, acc, sem):
    c = lax.axis_index("c")
    s = lax.axis_index("s")
    wid = c * _NSUB + s
    zs = buf.at[pl.ds(0, _RCH)]
    _fill(buf, _RCH, _DW, 0.0)
    for t in range(_NR):
      pltpu.sync_copy(zs, acc.at[pl.ds(_stripe(s, t), _RCH)])
    plsc.subcore_barrier()
    _fill(buf, _CH, _DW, 1.0)
    ones = buf.at[pl.ds(0, _CH)]
    pltpu.sync_copy(dst_hbm.at[wid], dst_v)

    @pl.loop(0, _NCHUNK)
    def _(j):
      pltpu.async_copy(ones, acc.at[dst_v.at[j]], sem, add=True)

    @pl.loop(0, _NCHUNK)
    def _(j):
      pltpu.make_async_copy(ones, acc.at[dst_v.at[j]], sem).wait()

    plsc.subcore_barrier()
    for t in range(_NR):
      rows = pl.ds(_stripe(s, t), _RCH)
      pltpu.sync_copy(acc.at[rows], zs)
      pltpu.sync_copy(zs, out_hbm.at[c, rows])

  return k


def _dinv(deg_ref):
  return lax.rsqrt(deg_ref[0, :, 0:1] + deg_ref[1, :, 0:1] + 1.0)


def _split2(z, o_ref):
  o_ref[0] = z[:, :64]
  o_ref[1] = z[:, 64:]


def _cat2(s_ref, g_ref):
  return jnp.concatenate([s_ref[0] + g_ref[0], s_ref[1] + g_ref[1]], axis=1)


def _k_g1(deg_ref, x_ref, o_ref):
  _split2(_dinv(deg_ref) * x_ref[...], o_ref)


def _k_first(deg_ref, s_ref, g_ref, w1_ref, b1_ref, w2_ref, o_ref):
  dinv = _dinv(deg_ref)
  p = dinv * _cat2(s_ref, g_ref)
  h = jnp.maximum(
      jnp.dot(p, w1_ref[...], preferred_element_type=jnp.float32) + b1_ref[...], 0.0)
  _split2(dinv * jnp.dot(h, w2_ref[...], preferred_element_type=jnp.float32), o_ref)


def _k_mid3(deg_ref, s_ref, g_ref, b_ref, w_ref, o_ref):
  dinv = _dinv(deg_ref)
  h = jnp.maximum(dinv * _cat2(s_ref, g_ref) + b_ref[...], 0.0)
  o_ref[...] = dinv * jnp.dot(h, w_ref[...], preferred_element_type=jnp.float32)


def _k_mid(deg_ref, s_ref, g_ref, b_ref, w_ref, o_ref):
  dinv = _dinv(deg_ref)
  h = jnp.maximum(dinv * (s_ref[0] + s_ref[1] + g_ref[...]) + b_ref[...], 0.0)
  o_ref[...] = dinv * jnp.dot(h, w_ref[...], preferred_element_type=jnp.float32)


def _k_last(deg_ref, s_ref, g_ref, b_ref, o_ref):
  dinv = _dinv(deg_ref)
  o_ref[...] = jnp.maximum(dinv * (s_ref[0] + s_ref[1] + g_ref[...]) + b_ref[...], 0.0)


_DEG_SPEC = pl.BlockSpec((2, _B, _DW), lambda i: (0, i, 0))


def _s_spec(p):
  return pl.BlockSpec((2, _B, p), lambda i: (0, i, 0))


def _g_spec(p):
  return pl.BlockSpec((_B, p), lambda i: (i, 0))


def _w_spec(p, q):
  return pl.BlockSpec((p, q), lambda i: (0, 0))


def _b_spec(q):
  return pl.BlockSpec((1, q), lambda i: (0, 0))


_SG_SPEC = pl.BlockSpec((2, _B, 64), lambda i: (0, i, 0))


def _pc(body, q, in_specs):
  return pl.pallas_call(
      body,
      grid=(_G,),
      in_specs=in_specs,
      out_specs=pl.BlockSpec((_B, q), lambda i: (i, 0)),
      out_shape=jax.ShapeDtypeStruct((_N, q), jnp.float32),
  )


def _pc2(body, in_specs):
  return pl.pallas_call(
      body,
      grid=(_G,),
      in_specs=in_specs,
      out_specs=_SG_SPEC,
      out_shape=jax.ShapeDtypeStruct((2, _N, 64), jnp.float32),
  )


@jax.jit
def kernel(x, edge_index, W1, b1, W2, b2, W3, b3, W4, b4, W5, b5):
  src = edge_index[0].reshape(_NW, _NCHUNK, _CH)
  dst = edge_index[1].reshape(_NW, _NCHUNK, _CH)
  ecs = (src, dst)

  deg = _deg()(dst)
  g1 = _pc2(_k_g1, [_DEG_SPEC, _g_spec(128)])(deg, x)
  S = _agg128cs()(g1, *ecs)
  g2 = _pc2(_k_first,
            [_DEG_SPEC, _SG_SPEC, _SG_SPEC, _w_spec(128, 256),
             _b_spec(256), _w_spec(256, 128)])(
                deg, S, g1, W1, b1.reshape(1, -1), W2)
  S = _agg128cs()(g2, *ecs)
  g3 = _pc(_k_mid3, 64,
           [_DEG_SPEC, _SG_SPEC, _SG_SPEC, _b_spec(128),
            _w_spec(128, 64)])(deg, S, g2, b2.reshape(1, -1), W3)
  S = _agg(64)(g3, src, dst)
  g4 = _pc(_k_mid, 32,
           [_DEG_SPEC, _s_spec(64), _g_spec(64), _b_spec(64),
            _w_spec(64, 32)])(deg, S, g3, b3.reshape(1, -1), W4)
  S = _agg(32)(g4, src, dst)
  g5 = _pc(_k_mid, 16,
           [_DEG_SPEC, _s_spec(32), _g_spec(32), _b_spec(32),
            _w_spec(32, 16)])(deg, S, g4, b4.reshape(1, -1), W5)
  S = _agg(16)(g5, src, dst)
  out = _pc(_k_last, 16, [_DEG_SPEC, _s_spec(16), _g_spec(16), _b_spec(16)])(
      deg, S, g5, b5.reshape(1, -1))
  return out


# speedup vs baseline: 1.0216x; 1.0063x over previous
import functools
import jax
import jax.numpy as jnp
from jax import lax
from jax.experimental import pallas as pl
from jax.experimental.pallas import tpu as pltpu
from jax.experimental.pallas import tpu_sc as plsc

_N = 10000
_E = 320000
_NCORE = 2
_NSUB = 16
_NW = _NCORE * _NSUB
_EPT = _E // _NW
_CH = 250
_NCHUNK = _EPT // _CH
_CS_NSTAGE = 2
_RCH = 128
_NR = 5

_B = 2000
_G = _N // _B


def _fill(ref, rows, d, val):
  v = jnp.full((16,), val, jnp.float32)

  @pl.loop(0, rows)
  def _(r):
    for c in range(d // 16):
      ref[r, pl.ds(c * 16, 16)] = v


def _stripe(s, t):
  off = jnp.minimum((s * _NR + t) * _RCH, _N - _RCH)
  return pl.multiple_of(off, _RCH)


@functools.cache
def _agg(d):
  ch = 500 if d <= 32 else _CH
  nc = _EPT // ch
  mesh = plsc.VectorSubcoreMesh(core_axis_name="c", subcore_axis_name="s")

  @functools.partial(
      pl.kernel,
      out_type=jax.ShapeDtypeStruct((_NCORE, _N, d), jnp.float32),
      mesh=mesh,
      compiler_params=pltpu.CompilerParams(use_tc_tiling_on_sc=False),
      scratch_types=(
          [pltpu.VMEM((nc, ch), jnp.int32),
           pltpu.VMEM((nc, ch), jnp.int32),
           pltpu.VMEM((max(ch, _RCH), d), jnp.float32)]
          + [pltpu.VMEM((ch, d), jnp.float32)] * 3
          + [pltpu.VMEM_SHARED((_N, d), jnp.float32)]
          + [pltpu.SemaphoreType.DMA] * 8
      ),
  )
  def k(g_hbm, src_hbm, dst_hbm, out_hbm, src_v, dst_v, buf, *rest):
    rest = list(rest)
    bufs = [buf.at[pl.ds(0, ch)]] + rest[:3]
    acc = rest[3]
    gs = rest[4:8]
    ss = rest[8:12]
    c = lax.axis_index("c")
    s = lax.axis_index("s")
    wid = c * _NSUB + s
    zs = buf.at[pl.ds(0, _RCH)]
    _fill(buf, _RCH, d, 0.0)
    for t in range(_NR):
      pltpu.sync_copy(zs, acc.at[pl.ds(_stripe(s, t), _RCH)])
    plsc.subcore_barrier()

    def gather(j, r):
      pltpu.async_copy(g_hbm.at[src_v.at[j]], bufs[r], gs[r])

    def gwait(j, r):
      pltpu.make_async_copy(g_hbm.at[src_v.at[j]], bufs[r], gs[r]).wait()

    def scat(j, r):
      pltpu.async_copy(bufs[r], acc.at[dst_v.at[j]], ss[r], add=True)

    def swait(j, r):
      pltpu.make_async_copy(bufs[r], acc.at[dst_v.at[j]], ss[r]).wait()

    pltpu.sync_copy(src_hbm.at[wid], src_v)
    pltpu.sync_copy(dst_hbm.at[wid], dst_v)
    gather(0, 0)
    gather(1, 1)
    gwait(0, 0); scat(0, 0); gather(2, 2)
    gwait(1, 1); scat(1, 1); gather(3, 3)
    gwait(2, 2); scat(2, 2); swait(0, 0); gather(4, 0)
    gwait(3, 3); scat(3, 3); swait(1, 1); gather(5, 1)

    @pl.loop(1, nc // 4)
    def _(bi):
      jb = bi * 4
      for ph in range(4):
        j = jb + ph
        r2 = (ph + 2) % 4
        gwait(j, ph)
        scat(j, ph)
        swait(j - 2, r2)
        nxt = jnp.minimum(j + 2, nc - 1)
        gather(nxt, r2)

    swait(nc - 2, 2)
    swait(nc - 1, 3)
    gwait(nc - 1, 0)
    gwait(nc - 1, 1)

    plsc.subcore_barrier()
    for t in range(_NR):
      rows = pl.ds(_stripe(s, t), _RCH)
      pltpu.sync_copy(acc.at[rows], zs)
      pltpu.sync_copy(zs, out_hbm.at[c, rows])

  return k


@functools.cache
def _agg128cs():
  ch, schunk = _CH, _NCHUNK
  mesh = plsc.VectorSubcoreMesh(core_axis_name="c", subcore_axis_name="s")

  @functools.partial(
      pl.kernel,
      out_type=jax.ShapeDtypeStruct((_NCORE, _N, 64), jnp.float32),
      mesh=mesh,
      compiler_params=pltpu.CompilerParams(use_tc_tiling_on_sc=False),
      scratch_types=(
          [pltpu.VMEM((schunk, ch), jnp.int32),
           pltpu.VMEM((schunk, ch), jnp.int32),
           pltpu.VMEM((ch, 64), jnp.float32)]
          + [pltpu.VMEM((ch, 64), jnp.float32)] * 3
          + [pltpu.VMEM_SHARED((_N, 64), jnp.float32)]
          + [pltpu.SemaphoreType.DMA] * 8
      ),
  )
  def k(g_hbm, src_hbm, dst_hbm, out_hbm, src_v, dst_v, buf, *rest):
    rest = list(rest)
    bufs = [buf] + rest[:3]
    acc = rest[3]
    gs = rest[4:8]
    ss = rest[8:12]
    c = lax.axis_index("c")
    s = lax.axis_index("s")
    g_c = g_hbm.at[c]
    zs = buf.at[pl.ds(0, _RCH)]
    _fill(buf, _RCH, 64, 0.0)
    for t in range(_NR):
      pltpu.sync_copy(zs, acc.at[pl.ds(_stripe(s, t), _RCH)])
    plsc.subcore_barrier()

    def gather(j, r):
      pltpu.async_copy(g_c.at[src_v.at[j]], bufs[r], gs[r])

    def gwait(j, r):
      pltpu.make_async_copy(g_c.at[src_v.at[j]], bufs[r], gs[r]).wait()

    def scat(j, r):
      pltpu.async_copy(bufs[r], acc.at[dst_v.at[j]], ss[r], add=True)

    def swait(j, r):
      pltpu.make_async_copy(bufs[r], acc.at[dst_v.at[j]], ss[r]).wait()

    for h in range(_CS_NSTAGE):
      pltpu.sync_copy(src_hbm.at[s + _NSUB * h], src_v)
      pltpu.sync_copy(dst_hbm.at[s + _NSUB * h], dst_v)
      gather(0, 0)
      gather(1, 1)
      gwait(0, 0); scat(0, 0); gather(2, 2)
      gwait(1, 1); scat(1, 1); gather(3, 3)
      gwait(2, 2); scat(2, 2); swait(0, 0); gather(4, 0)
      gwait(3, 3); scat(3, 3); swait(1, 1); gather(5, 1)

      @pl.loop(1, schunk // 4)
      def _(bi):
        jb = bi * 4
        for ph in range(4):
          j = jb + ph
          r2 = (ph + 2) % 4
          gwait(j, ph)
          scat(j, ph)
          swait(j - 2, r2)
          nxt = jnp.minimum(j + 2, schunk - 1)
          gather(nxt, r2)

      swait(schunk - 2, 2)
      swait(schunk - 1, 3)
      gwait(schunk - 1, 0)
      gwait(schunk - 1, 1)

    plsc.subcore_barrier()
    for t in range(_NR):
      rows = pl.ds(_stripe(s, t), _RCH)
      pltpu.sync_copy(acc.at[rows], zs)
      pltpu.sync_copy(zs, out_hbm.at[c, rows])

  return k


_DW = 16


@functools.cache
def _deg():
  mesh = plsc.VectorSubcoreMesh(core_axis_name="c", subcore_axis_name="s")

  @functools.partial(
      pl.kernel,
      out_type=jax.ShapeDtypeStruct((_NCORE, _N, _DW), jnp.float32),
      mesh=mesh,
      compiler_params=pltpu.CompilerParams(use_tc_tiling_on_sc=False),
      scratch_types=[
          pltpu.VMEM((_EPT // 500, 500), jnp.int32),
          pltpu.VMEM((500, _DW), jnp.float32),
          pltpu.VMEM_SHARED((_N, _DW), jnp.float32),
          pltpu.SemaphoreType.DMA,
      ],
  )
  def k(dst_hbm, out_hbm, dst_v, buf, acc, sem):
    c = lax.axis_index("c")
    s = lax.axis_index("s")
    wid = c * _NSUB + s
    zs = buf.at[pl.ds(0, _RCH)]
    _fill(buf, _RCH, _DW, 0.0)
    for t in range(_NR):
      pltpu.sync_copy(zs, acc.at[pl.ds(_stripe(s, t), _RCH)])
    plsc.subcore_barrier()
    _fill(buf, 500, _DW, 1.0)
    ones = buf.at[pl.ds(0, 500)]
    pltpu.sync_copy(dst_hbm.at[wid], dst_v)

    @pl.loop(0, _EPT // 500)
    def _(j):
      pltpu.async_copy(ones, acc.at[dst_v.at[j]], sem, add=True)

    @pl.loop(0, _EPT // 500)
    def _(j):
      pltpu.make_async_copy(ones, acc.at[dst_v.at[j]], sem).wait()

    plsc.subcore_barrier()
    for t in range(_NR):
      rows = pl.ds(_stripe(s, t), _RCH)
      pltpu.sync_copy(acc.at[rows], zs)
      pltpu.sync_copy(zs, out_hbm.at[c, rows])

  return k


def _dinv(deg_ref):
  return lax.rsqrt(deg_ref[0, :, 0:1] + deg_ref[1, :, 0:1] + 1.0)


def _split2(z, o_ref):
  o_ref[0] = z[:, :64]
  o_ref[1] = z[:, 64:]


def _cat2(s_ref, g_ref):
  return jnp.concatenate([s_ref[0] + g_ref[0], s_ref[1] + g_ref[1]], axis=1)


def _k_g1(deg_ref, x_ref, o_ref):
  _split2(_dinv(deg_ref) * x_ref[...], o_ref)


def _k_first(deg_ref, s_ref, g_ref, w1_ref, b1_ref, w2_ref, o_ref):
  dinv = _dinv(deg_ref)
  p = dinv * _cat2(s_ref, g_ref)
  h = jnp.maximum(
      jnp.dot(p, w1_ref[...], preferred_element_type=jnp.float32) + b1_ref[...], 0.0)
  _split2(dinv * jnp.dot(h, w2_ref[...], preferred_element_type=jnp.float32), o_ref)


def _k_mid3(deg_ref, s_ref, g_ref, b_ref, w_ref, o_ref):
  dinv = _dinv(deg_ref)
  h = jnp.maximum(dinv * _cat2(s_ref, g_ref) + b_ref[...], 0.0)
  o_ref[...] = dinv * jnp.dot(h, w_ref[...], preferred_element_type=jnp.float32)


def _k_mid(deg_ref, s_ref, g_ref, b_ref, w_ref, o_ref):
  dinv = _dinv(deg_ref)
  h = jnp.maximum(dinv * (s_ref[0] + s_ref[1] + g_ref[...]) + b_ref[...], 0.0)
  o_ref[...] = dinv * jnp.dot(h, w_ref[...], preferred_element_type=jnp.float32)


def _k_last(deg_ref, s_ref, g_ref, b_ref, o_ref):
  dinv = _dinv(deg_ref)
  o_ref[...] = jnp.maximum(dinv * (s_ref[0] + s_ref[1] + g_ref[...]) + b_ref[...], 0.0)


_DEG_SPEC = pl.BlockSpec((2, _B, _DW), lambda i: (0, i, 0))


def _s_spec(p):
  return pl.BlockSpec((2, _B, p), lambda i: (0, i, 0))


def _g_spec(p):
  return pl.BlockSpec((_B, p), lambda i: (i, 0))


def _w_spec(p, q):
  return pl.BlockSpec((p, q), lambda i: (0, 0))


def _b_spec(q):
  return pl.BlockSpec((1, q), lambda i: (0, 0))


_SG_SPEC = pl.BlockSpec((2, _B, 64), lambda i: (0, i, 0))


def _pc(body, q, in_specs):
  return pl.pallas_call(
      body,
      grid=(_G,),
      in_specs=in_specs,
      out_specs=pl.BlockSpec((_B, q), lambda i: (i, 0)),
      out_shape=jax.ShapeDtypeStruct((_N, q), jnp.float32),
  )


def _pc2(body, in_specs):
  return pl.pallas_call(
      body,
      grid=(_G,),
      in_specs=in_specs,
      out_specs=_SG_SPEC,
      out_shape=jax.ShapeDtypeStruct((2, _N, 64), jnp.float32),
  )


@jax.jit
def kernel(x, edge_index, W1, b1, W2, b2, W3, b3, W4, b4, W5, b5):
  src = edge_index[0].reshape(_NW, _NCHUNK, _CH)
  dst = edge_index[1].reshape(_NW, _NCHUNK, _CH)
  src5 = edge_index[0].reshape(_NW, _EPT // 500, 500)
  dst5 = edge_index[1].reshape(_NW, _EPT // 500, 500)
  ecs = (src, dst)

  deg = _deg()(dst5)
  g1 = _pc2(_k_g1, [_DEG_SPEC, _g_spec(128)])(deg, x)
  S = _agg128cs()(g1, *ecs)
  g2 = _pc2(_k_first,
            [_DEG_SPEC, _SG_SPEC, _SG_SPEC, _w_spec(128, 256),
             _b_spec(256), _w_spec(256, 128)])(
                deg, S, g1, W1, b1.reshape(1, -1), W2)
  S = _agg128cs()(g2, *ecs)
  g3 = _pc(_k_mid3, 64,
           [_DEG_SPEC, _SG_SPEC, _SG_SPEC, _b_spec(128),
            _w_spec(128, 64)])(deg, S, g2, b2.reshape(1, -1), W3)
  S = _agg(64)(g3, src, dst)
  g4 = _pc(_k_mid, 32,
           [_DEG_SPEC, _s_spec(64), _g_spec(64), _b_spec(64),
            _w_spec(64, 32)])(deg, S, g3, b3.reshape(1, -1), W4)
  S = _agg(32)(g4, src5, dst5)
  g5 = _pc(_k_mid, 16,
           [_DEG_SPEC, _s_spec(32), _g_spec(32), _b_spec(32),
            _w_spec(32, 16)])(deg, S, g4, b4.reshape(1, -1), W5)
  S = _agg(16)(g5, src5, dst5)
  out = _pc(_k_last, 16, [_DEG_SPEC, _s_spec(16), _g_spec(16), _b_spec(16)])(
      deg, S, g5, b5.reshape(1, -1))
  return out


# speedup vs baseline: 1.0340x; 1.0122x over previous
import functools
import jax
import jax.numpy as jnp
from jax import lax
from jax.experimental import pallas as pl
from jax.experimental.pallas import tpu as pltpu
from jax.experimental.pallas import tpu_sc as plsc

_N = 10000
_E = 320000
_NCORE = 2
_NSUB = 16
_NW = _NCORE * _NSUB
_EPT = _E // _NW
_CH = 250
_NCHUNK = _EPT // _CH
_CS_NSTAGE = 2
_RCH = 128
_NR = 5

_B = 2000
_G = _N // _B


def _fill(ref, rows, d, val):
  v = jnp.full((16,), val, jnp.float32)

  @pl.loop(0, rows)
  def _(r):
    for c in range(d // 16):
      ref[r, pl.ds(c * 16, 16)] = v


def _stripe(s, t):
  off = jnp.minimum((s * _NR + t) * _RCH, _N - _RCH)
  return pl.multiple_of(off, _RCH)


def _zero_acc(s, zs, acc, sem):
  for t in range(_NR):
    pltpu.async_copy(zs, acc.at[pl.ds(_stripe(s, t), _RCH)], sem)
  for t in range(_NR):
    pltpu.make_async_copy(zs, acc.at[pl.ds(_stripe(s, t), _RCH)], sem).wait()


def _drain_acc(s, c, acc, out_hbm, d0, d1, g0, g1, s0, s1):
  rows = [pl.ds(_stripe(s, t), _RCH) for t in range(_NR)]
  bufs = [d0, d1]
  gsem = [g0, g1]
  ssem = [s0, s1]

  pltpu.async_copy(acc.at[rows[0]], d0, g0)
  for t in range(_NR):
    p = t % 2
    pltpu.make_async_copy(acc.at[rows[t]], bufs[p], gsem[p]).wait()
    pltpu.async_copy(bufs[p], out_hbm.at[c, rows[t]], ssem[p])
    if t + 1 < _NR:
      q = (t + 1) % 2
      if t >= 1:
        pltpu.make_async_copy(bufs[q], out_hbm.at[c, rows[t - 1]], ssem[q]).wait()
      pltpu.async_copy(acc.at[rows[t + 1]], bufs[q], gsem[q])
  pltpu.make_async_copy(d0, out_hbm.at[c, rows[_NR - 1]], ssem[(_NR - 1) % 2]).wait()
  pltpu.make_async_copy(d1, out_hbm.at[c, rows[_NR - 2]], ssem[(_NR - 2) % 2]).wait()


@functools.cache
def _agg(d):
  ch = 500 if d <= 32 else _CH
  nc = _EPT // ch
  mesh = plsc.VectorSubcoreMesh(core_axis_name="c", subcore_axis_name="s")

  @functools.partial(
      pl.kernel,
      out_type=jax.ShapeDtypeStruct((_NCORE, _N, d), jnp.float32),
      mesh=mesh,
      compiler_params=pltpu.CompilerParams(use_tc_tiling_on_sc=False),
      scratch_types=(
          [pltpu.VMEM((nc, ch), jnp.int32),
           pltpu.VMEM((nc, ch), jnp.int32),
           pltpu.VMEM((max(ch, _RCH), d), jnp.float32)]
          + [pltpu.VMEM((ch, d), jnp.float32)] * 3
          + [pltpu.VMEM_SHARED((_N, d), jnp.float32)]
          + [pltpu.SemaphoreType.DMA] * 8
      ),
  )
  def k(g_hbm, src_hbm, dst_hbm, out_hbm, src_v, dst_v, buf, *rest):
    rest = list(rest)
    bufs = [buf.at[pl.ds(0, ch)]] + rest[:3]
    acc = rest[3]
    gs = rest[4:8]
    ss = rest[8:12]
    c = lax.axis_index("c")
    s = lax.axis_index("s")
    wid = c * _NSUB + s
    zs = buf.at[pl.ds(0, _RCH)]
    _fill(buf, _RCH, d, 0.0)
    _zero_acc(s, zs, acc, rest[4])
    plsc.subcore_barrier()

    def gather(j, r):
      pltpu.async_copy(g_hbm.at[src_v.at[j]], bufs[r], gs[r])

    def gwait(j, r):
      pltpu.make_async_copy(g_hbm.at[src_v.at[j]], bufs[r], gs[r]).wait()

    def scat(j, r):
      pltpu.async_copy(bufs[r], acc.at[dst_v.at[j]], ss[r], add=True)

    def swait(j, r):
      pltpu.make_async_copy(bufs[r], acc.at[dst_v.at[j]], ss[r]).wait()

    pltpu.sync_copy(src_hbm.at[wid], src_v)
    pltpu.sync_copy(dst_hbm.at[wid], dst_v)
    gather(0, 0)
    gather(1, 1)
    gwait(0, 0); scat(0, 0); gather(2, 2)
    gwait(1, 1); scat(1, 1); gather(3, 3)
    gwait(2, 2); scat(2, 2); swait(0, 0); gather(4, 0)
    gwait(3, 3); scat(3, 3); swait(1, 1); gather(5, 1)

    @pl.loop(1, nc // 4)
    def _(bi):
      jb = bi * 4
      for ph in range(4):
        j = jb + ph
        r2 = (ph + 2) % 4
        gwait(j, ph)
        scat(j, ph)
        swait(j - 2, r2)
        nxt = jnp.minimum(j + 2, nc - 1)
        gather(nxt, r2)

    swait(nc - 2, 2)
    swait(nc - 1, 3)
    gwait(nc - 1, 0)
    gwait(nc - 1, 1)

    plsc.subcore_barrier()
    _drain_acc(s, c, acc, out_hbm, rest[0].at[pl.ds(0, _RCH)],
               rest[1].at[pl.ds(0, _RCH)], gs[0], gs[1], ss[0], ss[1])

  return k


@functools.cache
def _agg128cs():
  ch, schunk = _CH, _NCHUNK
  mesh = plsc.VectorSubcoreMesh(core_axis_name="c", subcore_axis_name="s")

  @functools.partial(
      pl.kernel,
      out_type=jax.ShapeDtypeStruct((_NCORE, _N, 64), jnp.float32),
      mesh=mesh,
      compiler_params=pltpu.CompilerParams(use_tc_tiling_on_sc=False),
      scratch_types=(
          [pltpu.VMEM((schunk, ch), jnp.int32),
           pltpu.VMEM((schunk, ch), jnp.int32),
           pltpu.VMEM((ch, 64), jnp.float32)]
          + [pltpu.VMEM((ch, 64), jnp.float32)] * 3
          + [pltpu.VMEM_SHARED((_N, 64), jnp.float32)]
          + [pltpu.SemaphoreType.DMA] * 8
      ),
  )
  def k(g_hbm, src_hbm, dst_hbm, out_hbm, src_v, dst_v, buf, *rest):
    rest = list(rest)
    bufs = [buf] + rest[:3]
    acc = rest[3]
    gs = rest[4:8]
    ss = rest[8:12]
    c = lax.axis_index("c")
    s = lax.axis_index("s")
    g_c = g_hbm.at[c]
    zs = buf.at[pl.ds(0, _RCH)]
    _fill(buf, _RCH, 64, 0.0)
    _zero_acc(s, zs, acc, rest[4])
    plsc.subcore_barrier()

    def gather(j, r):
      pltpu.async_copy(g_c.at[src_v.at[j]], bufs[r], gs[r])

    def gwait(j, r):
      pltpu.make_async_copy(g_c.at[src_v.at[j]], bufs[r], gs[r]).wait()

    def scat(j, r):
      pltpu.async_copy(bufs[r], acc.at[dst_v.at[j]], ss[r], add=True)

    def swait(j, r):
      pltpu.make_async_copy(bufs[r], acc.at[dst_v.at[j]], ss[r]).wait()

    for h in range(_CS_NSTAGE):
      pltpu.sync_copy(src_hbm.at[s + _NSUB * h], src_v)
      pltpu.sync_copy(dst_hbm.at[s + _NSUB * h], dst_v)
      gather(0, 0)
      gather(1, 1)
      gwait(0, 0); scat(0, 0); gather(2, 2)
      gwait(1, 1); scat(1, 1); gather(3, 3)
      gwait(2, 2); scat(2, 2); swait(0, 0); gather(4, 0)
      gwait(3, 3); scat(3, 3); swait(1, 1); gather(5, 1)

      @pl.loop(1, schunk // 4)
      def _(bi):
        jb = bi * 4
        for ph in range(4):
          j = jb + ph
          r2 = (ph + 2) % 4
          gwait(j, ph)
          scat(j, ph)
          swait(j - 2, r2)
          nxt = jnp.minimum(j + 2, schunk - 1)
          gather(nxt, r2)

      swait(schunk - 2, 2)
      swait(schunk - 1, 3)
      gwait(schunk - 1, 0)
      gwait(schunk - 1, 1)

    plsc.subcore_barrier()
    _drain_acc(s, c, acc, out_hbm, rest[0].at[pl.ds(0, _RCH)],
               rest[1].at[pl.ds(0, _RCH)], gs[0], gs[1], ss[0], ss[1])

  return k


_DW = 16


@functools.cache
def _deg():
  mesh = plsc.VectorSubcoreMesh(core_axis_name="c", subcore_axis_name="s")

  @functools.partial(
      pl.kernel,
      out_type=jax.ShapeDtypeStruct((_NCORE, _N, _DW), jnp.float32),
      mesh=mesh,
      compiler_params=pltpu.CompilerParams(use_tc_tiling_on_sc=False),
      scratch_types=[
          pltpu.VMEM((_EPT // 500, 500), jnp.int32),
          pltpu.VMEM((500, _DW), jnp.float32),
          pltpu.VMEM_SHARED((_N, _DW), jnp.float32),
          pltpu.SemaphoreType.DMA,
          pltpu.SemaphoreType.DMA,
          pltpu.SemaphoreType.DMA,
          pltpu.SemaphoreType.DMA,
      ],
  )
  def k(dst_hbm, out_hbm, dst_v, buf, acc, sem, sem2, sem3, sem4):
    c = lax.axis_index("c")
    s = lax.axis_index("s")
    wid = c * _NSUB + s
    zs = buf.at[pl.ds(0, _RCH)]
    _fill(buf, _RCH, _DW, 0.0)
    _zero_acc(s, zs, acc, sem)
    plsc.subcore_barrier()
    _fill(buf, 500, _DW, 1.0)
    ones = buf.at[pl.ds(0, 500)]
    pltpu.sync_copy(dst_hbm.at[wid], dst_v)

    @pl.loop(0, _EPT // 500)
    def _(j):
      pltpu.async_copy(ones, acc.at[dst_v.at[j]], sem, add=True)

    @pl.loop(0, _EPT // 500)
    def _(j):
      pltpu.make_async_copy(ones, acc.at[dst_v.at[j]], sem).wait()

    plsc.subcore_barrier()
    _drain_acc(s, c, acc, out_hbm, buf.at[pl.ds(0, _RCH)],
               buf.at[pl.ds(_RCH, _RCH)], sem, sem2, sem3, sem4)

  return k


def _dinv(deg_ref):
  return lax.rsqrt(deg_ref[0, :, 0:1] + deg_ref[1, :, 0:1] + 1.0)


def _split2(z, o_ref):
  o_ref[0] = z[:, :64]
  o_ref[1] = z[:, 64:]


def _cat2(s_ref, g_ref):
  return jnp.concatenate([s_ref[0] + g_ref[0], s_ref[1] + g_ref[1]], axis=1)


def _k_g1(deg_ref, x_ref, o_ref):
  _split2(_dinv(deg_ref) * x_ref[...], o_ref)


def _k_first(deg_ref, s_ref, g_ref, w1_ref, b1_ref, w2_ref, o_ref):
  dinv = _dinv(deg_ref)
  p = dinv * _cat2(s_ref, g_ref)
  h = jnp.maximum(
      jnp.dot(p, w1_ref[...], preferred_element_type=jnp.float32) + b1_ref[...], 0.0)
  _split2(dinv * jnp.dot(h, w2_ref[...], preferred_element_type=jnp.float32), o_ref)


def _k_mid3(deg_ref, s_ref, g_ref, b_ref, w_ref, o_ref):
  dinv = _dinv(deg_ref)
  h = jnp.maximum(dinv * _cat2(s_ref, g_ref) + b_ref[...], 0.0)
  o_ref[...] = dinv * jnp.dot(h, w_ref[...], preferred_element_type=jnp.float32)


def _k_mid(deg_ref, s_ref, g_ref, b_ref, w_ref, o_ref):
  dinv = _dinv(deg_ref)
  h = jnp.maximum(dinv * (s_ref[0] + s_ref[1] + g_ref[...]) + b_ref[...], 0.0)
  o_ref[...] = dinv * jnp.dot(h, w_ref[...], preferred_element_type=jnp.float32)


def _k_last(deg_ref, s_ref, g_ref, b_ref, o_ref):
  dinv = _dinv(deg_ref)
  o_ref[...] = jnp.maximum(dinv * (s_ref[0] + s_ref[1] + g_ref[...]) + b_ref[...], 0.0)


_DEG_SPEC = pl.BlockSpec((2, _B, _DW), lambda i: (0, i, 0))


def _s_spec(p):
  return pl.BlockSpec((2, _B, p), lambda i: (0, i, 0))


def _g_spec(p):
  return pl.BlockSpec((_B, p), lambda i: (i, 0))


def _w_spec(p, q):
  return pl.BlockSpec((p, q), lambda i: (0, 0))


def _b_spec(q):
  return pl.BlockSpec((1, q), lambda i: (0, 0))


_SG_SPEC = pl.BlockSpec((2, _B, 64), lambda i: (0, i, 0))


def _pc(body, q, in_specs):
  return pl.pallas_call(
      body,
      grid=(_G,),
      in_specs=in_specs,
      out_specs=pl.BlockSpec((_B, q), lambda i: (i, 0)),
      out_shape=jax.ShapeDtypeStruct((_N, q), jnp.float32),
  )


def _pc2(body, in_specs):
  return pl.pallas_call(
      body,
      grid=(_G,),
      in_specs=in_specs,
      out_specs=_SG_SPEC,
      out_shape=jax.ShapeDtypeStruct((2, _N, 64), jnp.float32),
  )


@jax.jit
def kernel(x, edge_index, W1, b1, W2, b2, W3, b3, W4, b4, W5, b5):
  src = edge_index[0].reshape(_NW, _NCHUNK, _CH)
  dst = edge_index[1].reshape(_NW, _NCHUNK, _CH)
  src5 = edge_index[0].reshape(_NW, _EPT // 500, 500)
  dst5 = edge_index[1].reshape(_NW, _EPT // 500, 500)
  ecs = (src, dst)

  deg = _deg()(dst5)
  g1 = _pc2(_k_g1, [_DEG_SPEC, _g_spec(128)])(deg, x)
  S = _agg128cs()(g1, *ecs)
  g2 = _pc2(_k_first,
            [_DEG_SPEC, _SG_SPEC, _SG_SPEC, _w_spec(128, 256),
             _b_spec(256), _w_spec(256, 128)])(
                deg, S, g1, W1, b1.reshape(1, -1), W2)
  S = _agg128cs()(g2, *ecs)
  g3 = _pc(_k_mid3, 64,
           [_DEG_SPEC, _SG_SPEC, _SG_SPEC, _b_spec(128),
            _w_spec(128, 64)])(deg, S, g2, b2.reshape(1, -1), W3)
  S = _agg(64)(g3, src, dst)
  g4 = _pc(_k_mid, 32,
           [_DEG_SPEC, _s_spec(64), _g_spec(64), _b_spec(64),
            _w_spec(64, 32)])(deg, S, g3, b3.reshape(1, -1), W4)
  S = _agg(32)(g4, src5, dst5)
  g5 = _pc(_k_mid, 16,
           [_DEG_SPEC, _s_spec(32), _g_spec(32), _b_spec(32),
            _w_spec(32, 16)])(deg, S, g4, b4.reshape(1, -1), W5)
  S = _agg(16)(g5, src5, dst5)
  out = _pc(_k_last, 16, [_DEG_SPEC, _s_spec(16), _g_spec(16), _b_spec(16)])(
      deg, S, g5, b5.reshape(1, -1))
  return out
